# baseline (speedup 1.0000x reference)
import jax
import jax.numpy as jnp
from jax import lax
from jax.experimental import pallas as pl

_MC = 256
_ALPHA_C = 1.0


def _split3(v):
    h = v.astype(jnp.bfloat16)
    r = v - h.astype(jnp.float32)
    l = r.astype(jnp.bfloat16)
    l2 = (r - l.astype(jnp.float32)).astype(jnp.bfloat16)
    return h, l, l2


def _aug_pair(v):
    V = v.shape[0]
    vb = v.astype(jnp.bfloat16)
    n2 = jnp.sum(v * v, axis=1, keepdims=True)
    nh, nl, nl2 = _split3(n2)
    ones = jnp.ones((V, 3), jnp.bfloat16)
    zeros = jnp.zeros((V, 7), jnp.bfloat16)
    lhs = jnp.concatenate([vb, nh, nl, nl2, ones, zeros], axis=1)
    rhs = jnp.concatenate([-2.0 * vb, ones, nh, nl, nl2, zeros], axis=1)
    return lhs, rhs


def _chamfer_body(x_ref, y_ref, mask_ref, conf_ref,
                  conf_out, pred_out, gt_out):
    m = mask_ref[0]
    V2 = x_ref.shape[1]
    V1 = y_ref.shape[1]
    x = x_ref[0] * m.reshape(V2, 1)
    y = y_ref[0]

    x_lhs, x_rhs = _aug_pair(x)
    y_lhs, y_rhs = _aug_pair(y)

    dn = (((1,), (1,)), ((), ()))

    for j in range(V2 // _MC):
        sl = slice(j * _MC, (j + 1) * _MC)
        dj = lax.dot_general(y_lhs, x_rhs[sl, :], dn,
                             preferred_element_type=jnp.float32)
        cmin = jnp.maximum(jnp.min(dj, axis=0, keepdims=True), 0.0)
        lp = jnp.sqrt(cmin) * 100.0
        mj = m[:, sl]
        cj = conf_ref[0, :, sl]
        pred_out[0, :, sl] = lp * mj
        conf_out[0, :, sl] = (lp * cj - _ALPHA_C * jnp.log(cj)) * mj

    for j in range(V1 // _MC):
        sl = slice(j * _MC, (j + 1) * _MC)
        dj = lax.dot_general(x_lhs, y_rhs[sl, :], dn,
                             preferred_element_type=jnp.float32)
        cmin = jnp.maximum(jnp.min(dj, axis=0, keepdims=True), 0.0)
        gt_out[0, :, sl] = jnp.sqrt(cmin) * 100.0


def kernel(x_gt, x_pred, mask, confidence):
    B, V1, _ = x_gt.shape
    V2 = x_pred.shape[1]
    mask3 = mask.reshape(B, 1, V2)
    conf3 = confidence.reshape(B, 1, V2)

    conf_o, pred_o, gt_o = pl.pallas_call(
        _chamfer_body,
        grid=(B,),
        in_specs=[
            pl.BlockSpec((1, V2, 3), lambda b: (b, 0, 0)),
            pl.BlockSpec((1, V1, 3), lambda b: (b, 0, 0)),
            pl.BlockSpec((1, 1, V2), lambda b: (b, 0, 0)),
            pl.BlockSpec((1, 1, V2), lambda b: (b, 0, 0)),
        ],
        out_specs=[
            pl.BlockSpec((1, 1, V2), lambda b: (b, 0, 0)),
            pl.BlockSpec((1, 1, V2), lambda b: (b, 0, 0)),
            pl.BlockSpec((1, 1, V1), lambda b: (b, 0, 0)),
        ],
        out_shape=[
            jax.ShapeDtypeStruct((B, 1, V2), jnp.float32),
            jax.ShapeDtypeStruct((B, 1, V2), jnp.float32),
            jax.ShapeDtypeStruct((B, 1, V1), jnp.float32),
        ],
    )(x_pred, x_gt, mask3, conf3)

    return (conf_o.reshape(B, V2), pred_o.reshape(B, V2), gt_o.reshape(B, V1))


# speedup vs baseline: 1.5748x; 1.5748x over previous
import jax
import jax.numpy as jnp
from jax import lax
from jax.experimental import pallas as pl
from jax.experimental.pallas import tpu as pltpu

_MC = 256
_ALPHA_C = 1.0


def _split3(v):
    h = v.astype(jnp.bfloat16)
    r = v - h.astype(jnp.float32)
    l = r.astype(jnp.bfloat16)
    l2 = (r - l.astype(jnp.float32)).astype(jnp.bfloat16)
    return h, l, l2


def _chamfer_body(x_ref, y_ref, mask_ref, conf_ref,
                  conf_out, pred_out, gt_out, rowacc):
    B = x_ref.shape[0]
    V2 = x_ref.shape[2]
    V1 = y_ref.shape[2]
    ones3x = jnp.ones((3, V2), jnp.bfloat16)
    ones3y = jnp.ones((3, V1), jnp.bfloat16)
    zeros7x = jnp.zeros((7, V2), jnp.bfloat16)
    zeros7y = jnp.zeros((7, V1), jnp.bfloat16)
    dn = (((0,), (0,)), ((), ()))

    for b in range(B):
        m = mask_ref[b]
        x = x_ref[b] * m
        y = y_ref[b]

        x2 = jnp.sum(x * x, axis=0, keepdims=True)
        xh, xl, xl2 = _split3(x2)
        x_rhs = jnp.concatenate(
            [-2.0 * x.astype(jnp.bfloat16), ones3x, xh, xl, xl2, zeros7x], 0)

        y2 = jnp.sum(y * y, axis=0, keepdims=True)
        yh, yl, yl2 = _split3(y2)
        y_lhs = jnp.concatenate(
            [y.astype(jnp.bfloat16), yh, yl, yl2, ones3y, zeros7y], 0)

        rowacc[...] = jnp.full((V1, 1), jnp.inf, jnp.float32)

        for j in range(V2 // _MC):
            sl = slice(j * _MC, (j + 1) * _MC)
            dj = lax.dot_general(y_lhs, x_rhs[:, sl], dn,
                                 preferred_element_type=jnp.float32)
            cmin = jnp.maximum(jnp.min(dj, axis=0, keepdims=True), 0.0)
            lp = jnp.sqrt(cmin) * 100.0
            mj = m[:, sl]
            cj = conf_ref[b, :, sl]
            pred_out[b, :, sl] = lp * mj
            conf_out[b, :, sl] = (lp * cj - _ALPHA_C * jnp.log(cj)) * mj
            rowacc[...] = jnp.minimum(rowacc[...],
                                      jnp.min(dj, axis=1, keepdims=True))

        rmin = jnp.maximum(rowacc[...], 0.0)
        gt_out[b] = jnp.transpose(jnp.sqrt(rmin) * 100.0, (1, 0))


def kernel(x_gt, x_pred, mask, confidence):
    B, V1, _ = x_gt.shape
    V2 = x_pred.shape[1]
    x_t = jnp.transpose(x_pred, (0, 2, 1))
    y_t = jnp.transpose(x_gt, (0, 2, 1))
    mask3 = mask.reshape(B, 1, V2)
    conf3 = confidence.reshape(B, 1, V2)

    conf_o, pred_o, gt_o = pl.pallas_call(
        _chamfer_body,
        grid=(1,),
        in_specs=[
            pl.BlockSpec((B, 3, V2), lambda i: (0, 0, 0)),
            pl.BlockSpec((B, 3, V1), lambda i: (0, 0, 0)),
            pl.BlockSpec((B, 1, V2), lambda i: (0, 0, 0)),
            pl.BlockSpec((B, 1, V2), lambda i: (0, 0, 0)),
        ],
        out_specs=[
            pl.BlockSpec((B, 1, V2), lambda i: (0, 0, 0)),
            pl.BlockSpec((B, 1, V2), lambda i: (0, 0, 0)),
            pl.BlockSpec((B, 1, V1), lambda i: (0, 0, 0)),
        ],
        out_shape=[
            jax.ShapeDtypeStruct((B, 1, V2), jnp.float32),
            jax.ShapeDtypeStruct((B, 1, V2), jnp.float32),
            jax.ShapeDtypeStruct((B, 1, V1), jnp.float32),
        ],
        scratch_shapes=[pltpu.VMEM((V1, 1), jnp.float32)],
    )(x_t, y_t, mask3, conf3)

    return (conf_o.reshape(B, V2), pred_o.reshape(B, V2), gt_o.reshape(B, V1))


# speedup vs baseline: 1.7545x; 1.1141x over previous
import jax
import jax.numpy as jnp
from jax import lax
from jax.experimental import pallas as pl
from jax.experimental.pallas import tpu as pltpu

_MC = 256
_ALPHA_C = 1.0


def _split3(v):
    h = v.astype(jnp.bfloat16)
    r = v - h.astype(jnp.float32)
    l = r.astype(jnp.bfloat16)
    l2 = (r - l.astype(jnp.float32)).astype(jnp.bfloat16)
    return h, l, l2


def _chamfer_body(xb_ref, yb_ref, x2_ref, y2_ref, mask_ref, conf_ref,
                  conf_out, pred_out, gt_out, minacc):
    B = xb_ref.shape[0]
    V2 = xb_ref.shape[2]
    V1 = yb_ref.shape[2]
    ones3x = jnp.ones((3, V2), jnp.bfloat16)
    ones3y = jnp.ones((3, V1), jnp.bfloat16)
    zeros7x = jnp.zeros((7, V2), jnp.bfloat16)
    zeros7y = jnp.zeros((7, V1), jnp.bfloat16)
    dn = (((0,), (0,)), ((), ()))

    for b in range(B):
        m = mask_ref[b]

        xh, xl, xl2 = _split3(x2_ref[b])
        x_rhs = jnp.concatenate(
            [-2.0 * xb_ref[b], ones3x, xh, xl, xl2, zeros7x], 0)

        yh, yl, yl2 = _split3(y2_ref[b])
        y_lhs = jnp.concatenate(
            [yb_ref[b], yh, yl, yl2, ones3y, zeros7y], 0)

        for j in range(V2 // _MC):
            sl = slice(j * _MC, (j + 1) * _MC)
            dj = lax.dot_general(y_lhs, x_rhs[:, sl], dn,
                                 preferred_element_type=jnp.float32)
            cmin = jnp.maximum(jnp.min(dj, axis=0, keepdims=True), 0.0)
            lp = jnp.sqrt(cmin) * 100.0
            mj = m[:, sl]
            cj = conf_ref[b, :, sl]
            pred_out[b, :, sl] = lp * mj
            conf_out[b, :, sl] = (lp * cj - _ALPHA_C * jnp.log(cj)) * mj
            if j == 0:
                minacc[...] = dj
            else:
                minacc[...] = jnp.minimum(minacc[...], dj)

        rmin = jnp.maximum(
            jnp.min(minacc[...], axis=1, keepdims=True), 0.0)
        gt_out[b] = jnp.transpose(jnp.sqrt(rmin) * 100.0, (1, 0))


def kernel(x_gt, x_pred, mask, confidence):
    B, V1, _ = x_gt.shape
    V2 = x_pred.shape[1]
    xp = x_pred * mask[..., None]
    x2 = jnp.sum(xp * xp, axis=2).reshape(B, 1, V2)
    y2 = jnp.sum(x_gt * x_gt, axis=2).reshape(B, 1, V1)
    xb_t = jnp.transpose(xp.astype(jnp.bfloat16), (0, 2, 1))
    yb_t = jnp.transpose(x_gt.astype(jnp.bfloat16), (0, 2, 1))
    mask3 = mask.reshape(B, 1, V2)
    conf3 = confidence.reshape(B, 1, V2)

    full = lambda i: (0, 0, 0)
    conf_o, pred_o, gt_o = pl.pallas_call(
        _chamfer_body,
        grid=(1,),
        in_specs=[
            pl.BlockSpec((B, 3, V2), full),
            pl.BlockSpec((B, 3, V1), full),
            pl.BlockSpec((B, 1, V2), full),
            pl.BlockSpec((B, 1, V1), full),
            pl.BlockSpec((B, 1, V2), full),
            pl.BlockSpec((B, 1, V2), full),
        ],
        out_specs=[
            pl.BlockSpec((B, 1, V2), full),
            pl.BlockSpec((B, 1, V2), full),
            pl.BlockSpec((B, 1, V1), full),
        ],
        out_shape=[
            jax.ShapeDtypeStruct((B, 1, V2), jnp.float32),
            jax.ShapeDtypeStruct((B, 1, V2), jnp.float32),
            jax.ShapeDtypeStruct((B, 1, V1), jnp.float32),
        ],
        scratch_shapes=[pltpu.VMEM((V1, _MC), jnp.float32)],
    )(xb_t, yb_t, x2, y2, mask3, conf3)

    return (conf_o.reshape(B, V2), pred_o.reshape(B, V2), gt_o.reshape(B, V1))


# speedup vs baseline: 1.8996x; 1.0827x over previous
import jax
import jax.numpy as jnp
from jax import lax
from jax.experimental import pallas as pl
from jax.experimental.pallas import tpu as pltpu

_MC = 256
_ALPHA_C = 1.0


def _split3(v):
    h = v.astype(jnp.bfloat16)
    r = v - h.astype(jnp.float32)
    l = r.astype(jnp.bfloat16)
    l2 = (r - l.astype(jnp.float32)).astype(jnp.bfloat16)
    return h, l, l2


def _chamfer_body(pts_ref, n2_ref, mask_ref, conf_ref,
                  conf_out, pred_out, gt_out, minacc):
    B = pts_ref.shape[0]
    VT = pts_ref.shape[2]
    V2 = mask_ref.shape[2]
    V1 = VT - V2
    ones3x = jnp.ones((3, V2), jnp.bfloat16)
    ones3y = jnp.ones((3, V1), jnp.bfloat16)
    zeros7x = jnp.zeros((7, V2), jnp.bfloat16)
    zeros7y = jnp.zeros((7, V1), jnp.bfloat16)
    dn = (((0,), (0,)), ((), ()))

    for b in range(B):
        m = mask_ref[b]

        xh, xl, xl2 = _split3(n2_ref[b, :, :V2])
        x_rhs = jnp.concatenate(
            [-2.0 * pts_ref[b, :, :V2], ones3x, xh, xl, xl2, zeros7x], 0)

        yh, yl, yl2 = _split3(n2_ref[b, :, V2:])
        y_lhs = jnp.concatenate(
            [pts_ref[b, :, V2:], yh, yl, yl2, ones3y, zeros7y], 0)

        for j in range(V2 // _MC):
            sl = slice(j * _MC, (j + 1) * _MC)
            dj = lax.dot_general(y_lhs, x_rhs[:, sl], dn,
                                 preferred_element_type=jnp.float32)
            cmin = jnp.maximum(jnp.min(dj, axis=0, keepdims=True), 0.0)
            lp = jnp.sqrt(cmin) * 100.0
            mj = m[:, sl]
            cj = conf_ref[b, :, sl]
            pred_out[b, :, sl] = lp * mj
            conf_out[b, :, sl] = (lp * cj - _ALPHA_C * jnp.log(cj)) * mj
            if j == 0:
                minacc[...] = dj
            else:
                minacc[...] = jnp.minimum(minacc[...], dj)

        rmin = jnp.maximum(
            jnp.min(minacc[...], axis=1, keepdims=True), 0.0)
        gt_out[b] = jnp.transpose(jnp.sqrt(rmin) * 100.0, (1, 0))


def kernel(x_gt, x_pred, mask, confidence):
    B, V1, _ = x_gt.shape
    V2 = x_pred.shape[1]
    xp = x_pred * mask[..., None]
    pts = jnp.concatenate([xp, x_gt], axis=1)
    n2 = jnp.sum(pts * pts, axis=2).reshape(B, 1, V2 + V1)
    pts_t = jnp.transpose(pts.astype(jnp.bfloat16), (0, 2, 1))
    mask3 = mask.reshape(B, 1, V2)
    conf3 = confidence.reshape(B, 1, V2)

    full = lambda i: (0, 0, 0)
    conf_o, pred_o, gt_o = pl.pallas_call(
        _chamfer_body,
        grid=(1,),
        in_specs=[
            pl.BlockSpec((B, 3, V2 + V1), full),
            pl.BlockSpec((B, 1, V2 + V1), full),
            pl.BlockSpec((B, 1, V2), full),
            pl.BlockSpec((B, 1, V2), full),
        ],
        out_specs=[
            pl.BlockSpec((B, 1, V2), full),
            pl.BlockSpec((B, 1, V2), full),
            pl.BlockSpec((B, 1, V1), full),
        ],
        out_shape=[
            jax.ShapeDtypeStruct((B, 1, V2), jnp.float32),
            jax.ShapeDtypeStruct((B, 1, V2), jnp.float32),
            jax.ShapeDtypeStruct((B, 1, V1), jnp.float32),
        ],
        scratch_shapes=[pltpu.VMEM((V1, _MC), jnp.float32)],
    )(pts_t, n2, mask3, conf3)

    return (conf_o.reshape(B, V2), pred_o.reshape(B, V2), gt_o.reshape(B, V1))
